# baseline (speedup 1.0000x reference)
import functools

import numpy as np
import jax
import jax.numpy as jnp
from jax import lax
from jax.experimental import pallas as pl
from jax.experimental.pallas import tpu as pltpu
from jax.experimental.pallas import tpu_sc as plsc

_NEG = 0.2
_NC, _NS, _L = 2, 16, 16
_NW = _NC * _NS


def _lrelu(t):
    return jnp.maximum(t, _NEG * t)


def _prep_body(x_ref, wcat_ref, hsrc_ref, adp_ref, m1_ref):
    c1 = hsrc_ref.shape[1] - _L
    xw = jnp.dot(x_ref[...], wcat_ref[...], preferred_element_type=jnp.float32)
    asp = xw[:, c1:c1 + _L]
    adp = xw[:, c1 + _L:]
    hsrc_ref[:, :c1] = xw[:, :c1]
    hsrc_ref[:, c1:] = asp
    adp_ref[...] = adp
    m1_ref[...] = (jnp.max(asp, axis=0, keepdims=True)
                   + jnp.max(adp, axis=0, keepdims=True))


def _mid_body(p0_ref, p1_ref, hsrc_ref, adp_ref, m1_ref, b1p_ref, wcat2_ref,
              l2s_ref, l2d_ref, m2_ref):
    c1 = hsrc_ref.shape[1] - _L
    tot = p0_ref[...] + p1_ref[...]
    asp = hsrc_ref[:, c1:]
    wself = jnp.exp(_lrelu(asp + adp_ref[...]) - m1_ref[...])
    rr = lax.broadcasted_iota(jnp.int32, (_L, c1), 0)
    cc = lax.broadcasted_iota(jnp.int32, (_L, c1), 1)
    tile16 = (cc % _L == rr).astype(jnp.float32)
    wrep = jnp.dot(wself, tile16, preferred_element_type=jnp.float32)
    num = tot[:, :c1] + wrep * hsrc_ref[:, :c1]
    den = tot[:, c1:] + wself
    denrep = jnp.dot(den, tile16, preferred_element_type=jnp.float32)
    xh = num / (denrep + 1e-16) + b1p_ref[...]
    z = jnp.dot(xh, wcat2_ref[...], preferred_element_type=jnp.float32)
    h2 = z[:, :_L]
    as2 = z[:, _L:2 * _L]
    ad2 = z[:, 2 * _L:]
    m2_ref[...] = (jnp.max(as2, axis=0, keepdims=True)
                   + jnp.max(ad2, axis=0, keepdims=True))
    lanes = lax.broadcasted_iota(jnp.int32, (1, _L), 1)
    e0 = (lanes == 0).astype(jnp.float32)
    e1 = (lanes == 1).astype(jnp.float32)
    l2s_ref[:, :_L] = h2 * e0 + e1
    l2s_ref[:, _L:] = as2
    l2d_ref[...] = ad2


def _fin_body(q0_ref, q1_ref, l2s_ref, l2d_ref, m2_ref, b2_ref, out_ref):
    s1 = l2s_ref[:, :_L]
    s0 = l2s_ref[:, _L:]
    w = jnp.exp(_lrelu(s0 + l2d_ref[...]) - m2_ref[...])
    tot = (q0_ref[:, :_L] + q1_ref[:, :_L] + w * s1)
    rr = lax.broadcasted_iota(jnp.int32, (_L, _L), 0)
    sel0 = (rr == 0).astype(jnp.float32)
    sel1 = (rr == 1).astype(jnp.float32)
    numb = jnp.dot(tot, sel0, preferred_element_type=jnp.float32)
    denb = jnp.dot(tot, sel1, preferred_element_type=jnp.float32)
    o = numb / (denb + 1e-16) + b2_ref[...]
    out_ref[...] = jnp.where(o > 0, o, jnp.exp(jnp.minimum(o, 0.0)) - 1.0)


def _tc(body, out_shape, *ins):
    return pl.pallas_call(body, out_shape=out_shape)(*ins)


def _make_edge_pass(n_nodes, n_edges, row_w, n_vregs_payload, ch):
    ew = n_edges // _NW
    nch = ew // ch
    assert ew % ch == 0 and ch % 8 == 0 and ch <= 128
    npad = -(-n_nodes // (128 * _NS)) * (128 * _NS)
    rpt = npad // _NS
    assert rpt % ch == 0
    mesh = plsc.VectorSubcoreMesh(core_axis_name="c", subcore_axis_name="s",
                                  num_cores=_NC, num_subcores=_NS)

    nb = 5
    assert nch % nb == 0
    ngrp = nch // nb

    @functools.partial(
        pl.kernel,
        out_type=jax.ShapeDtypeStruct((_NC, npad, row_w), jnp.float32),
        mesh=mesh,
        compiler_params=pltpu.CompilerParams(use_tc_tiling_on_sc=False),
        scratch_types=[
            pltpu.VMEM((nb, ch), jnp.int32),
            pltpu.VMEM((nb, ch), jnp.int32),
            pltpu.VMEM((nb, ch, row_w), jnp.float32),
            pltpu.VMEM((nb, ch, _L), jnp.float32),
            pltpu.VMEM((_L,), jnp.float32),
            pltpu.VMEM_SHARED((npad, row_w), jnp.float32),
            pltpu.SemaphoreType.DMA((nb,)),
            pltpu.SemaphoreType.DMA((nb,)),
            pltpu.SemaphoreType.DMA,
        ],
    )
    def k(src_hbm, dst_hbm, pack_hbm, adp_hbm, m_hbm, out_hbm,
          idxs_v, idxd_v, rows_v, ad_v, m_v, acc_sh,
          semi, semg, sems):
        cid = lax.axis_index("c")
        sid = lax.axis_index("s")

        def zrow(r, carry):
            for kk in range(row_w // _L):
                rows_v[0, r, pl.ds(kk * _L, _L)] = jnp.zeros((_L,), jnp.float32)
            return carry
        lax.fori_loop(0, ch, zrow, 0)
        dz = []
        for b in range(rpt // ch):
            dz.append(pltpu.async_copy(
                rows_v.at[0], acc_sh.at[pl.ds(sid * rpt + b * ch, ch)], sems))
        for d in dz:
            d.wait()
        pltpu.sync_copy(m_hbm, m_v)
        plsc.subcore_barrier()

        mvec = m_v[...]
        ebase = (cid * _NS + sid) * ew

        def group(g, carry):
            base0 = ebase + g * (nb * ch)
            di = []
            for b in range(nb):
                di.append(pltpu.async_copy(
                    src_hbm.at[pl.ds(base0 + b * ch, ch)],
                    idxs_v.at[b], semi.at[b]))
                di.append(pltpu.async_copy(
                    dst_hbm.at[pl.ds(base0 + b * ch, ch)],
                    idxd_v.at[b], semi.at[b]))
            dg = []
            for b in range(nb):
                di[2 * b].wait()
                di[2 * b + 1].wait()
                dg.append(pltpu.async_copy(
                    pack_hbm.at[idxs_v.at[b]], rows_v.at[b], semg.at[b]))
                dg.append(pltpu.async_copy(
                    adp_hbm.at[idxd_v.at[b]], ad_v.at[b], semg.at[b]))
            dsc = []
            for b in range(nb):
                dg[2 * b].wait()
                dg[2 * b + 1].wait()
                def edge(e2, c2, b=b):
                    asv = rows_v[b, e2, pl.ds(row_w - _L, _L)]
                    t = asv + ad_v[b, e2, pl.ds(0, _L)]
                    w = jnp.exp(jnp.maximum(t, _NEG * t) - mvec)
                    for kk in range(n_vregs_payload):
                        hv = rows_v[b, e2, pl.ds(kk * _L, _L)]
                        rows_v[b, e2, pl.ds(kk * _L, _L)] = hv * w
                    rows_v[b, e2, pl.ds(row_w - _L, _L)] = w
                    return c2
                lax.fori_loop(0, ch, edge, 0, unroll=8)
                dsc.append(pltpu.async_copy(
                    rows_v.at[b], acc_sh.at[idxd_v.at[b]], sems, add=True))
            for d in dsc:
                d.wait()
            return carry
        lax.fori_loop(0, ngrp, group, 0)

        plsc.subcore_barrier()
        pltpu.sync_copy(acc_sh.at[pl.ds(sid * rpt, rpt)],
                        out_hbm.at[cid, pl.ds(sid * rpt, rpt)])
    return k


def kernel(x, edge_index, W1, a1s, a1d, b1, W2, a2s, a2d, b2):
    n, f_in = x.shape
    e = edge_index.shape[1]
    h, f_hid = a1s.shape
    c1 = h * f_hid
    p1 = c1 + _L
    f32 = jnp.float32

    hh, ff = np.meshgrid(np.arange(h), np.arange(f_hid), indexing="ij")
    dest = ((ff // 2) * _L + (ff % 2) * h + hh).reshape(-1)
    inv = np.empty((c1,), np.int64)
    inv[dest] = np.arange(c1)

    w1flat = jnp.transpose(W1, (1, 0, 2)).reshape(f_in, c1)
    w1perm = w1flat[:, inv]
    avs = jnp.einsum("hif,hf->ih", W1, a1s)
    avd = jnp.einsum("hif,hf->ih", W1, a1d)
    wcat = jnp.concatenate(
        [w1perm, jnp.tile(avs, (1, 2)), jnp.tile(avd, (1, 2))], axis=1)

    b1p = b1.reshape(1, c1)[:, inv]
    w2p = W2[inv, :]
    w2rep = jnp.tile(w2p, (1, _L))
    wcat2 = jnp.concatenate(
        [w2rep, w2rep * a2s[0], w2rep * a2d[0]], axis=1)

    src = edge_index[0]
    dst = edge_index[1]

    hsrc, adp, m1 = _tc(
        _prep_body,
        (jax.ShapeDtypeStruct((n, p1), f32),
         jax.ShapeDtypeStruct((n, _L), f32),
         jax.ShapeDtypeStruct((1, _L), f32)),
        x, wcat)

    part1 = _make_edge_pass(n, e, p1, c1 // _L, 40)(
        src, dst, hsrc, adp, m1.reshape(_L))

    l2s, l2d, m2 = _tc(
        _mid_body,
        (jax.ShapeDtypeStruct((n, 2 * _L), f32),
         jax.ShapeDtypeStruct((n, _L), f32),
         jax.ShapeDtypeStruct((1, _L), f32)),
        part1[0, :n], part1[1, :n], hsrc, adp, m1, b1p, wcat2)

    part2 = _make_edge_pass(n, e, 2 * _L, 1, 80)(
        src, dst, l2s, l2d, m2.reshape(_L))

    out16 = _tc(
        _fin_body,
        jax.ShapeDtypeStruct((n, _L), f32),
        part2[0, :n], part2[1, :n], l2s, l2d, m2, b2.reshape(1, 1))

    return out16[:, :1]


# speedup vs baseline: 1.3467x; 1.3467x over previous
import functools

import numpy as np
import jax
import jax.numpy as jnp
from jax import lax
from jax.experimental import pallas as pl
from jax.experimental.pallas import tpu as pltpu
from jax.experimental.pallas import tpu_sc as plsc

_NEG = 0.2
_NC, _NS, _L = 2, 16, 16
_NW = _NC * _NS


def _lrelu(t):
    return jnp.maximum(t, _NEG * t)


def _prep_body(x_ref, wcat_ref, hsrc_ref, adp_ref, m1_ref):
    c1 = hsrc_ref.shape[1] - _L
    xw = jnp.dot(x_ref[...], wcat_ref[...], preferred_element_type=jnp.float32)
    asp = xw[:, c1:c1 + _L]
    adp = xw[:, c1 + _L:]
    hsrc_ref[:, :c1] = xw[:, :c1]
    hsrc_ref[:, c1:] = asp
    adp_ref[...] = adp
    m1_ref[...] = (jnp.max(asp, axis=0, keepdims=True)
                   + jnp.max(adp, axis=0, keepdims=True))


def _mid_body(p0_ref, p1_ref, hsrc_ref, adp_ref, m1_ref, b1p_ref, wcat2_ref,
              l2s_ref, l2d_ref, m2_ref):
    c1 = hsrc_ref.shape[1] - _L
    tot = p0_ref[...] + p1_ref[...]
    asp = hsrc_ref[:, c1:]
    wself = jnp.exp(_lrelu(asp + adp_ref[...]) - m1_ref[...])
    rr = lax.broadcasted_iota(jnp.int32, (_L, c1), 0)
    cc = lax.broadcasted_iota(jnp.int32, (_L, c1), 1)
    tile16 = (cc % _L == rr).astype(jnp.float32)
    wrep = jnp.dot(wself, tile16, preferred_element_type=jnp.float32)
    num = tot[:, :c1] + wrep * hsrc_ref[:, :c1]
    den = tot[:, c1:] + wself
    denrep = jnp.dot(den, tile16, preferred_element_type=jnp.float32)
    xh = num / (denrep + 1e-16) + b1p_ref[...]
    z = jnp.dot(xh, wcat2_ref[...], preferred_element_type=jnp.float32)
    h2 = z[:, :_L]
    as2 = z[:, _L:2 * _L]
    ad2 = z[:, 2 * _L:]
    m2_ref[...] = (jnp.max(as2, axis=0, keepdims=True)
                   + jnp.max(ad2, axis=0, keepdims=True))
    lanes = lax.broadcasted_iota(jnp.int32, (1, _L), 1)
    e0 = (lanes == 0).astype(jnp.float32)
    e1 = (lanes == 1).astype(jnp.float32)
    l2s_ref[:, :_L] = h2 * e0 + e1
    l2s_ref[:, _L:] = as2
    l2d_ref[...] = ad2


def _fin_body(q0_ref, q1_ref, l2s_ref, l2d_ref, m2_ref, b2_ref, out_ref):
    s1 = l2s_ref[:, :_L]
    s0 = l2s_ref[:, _L:]
    w = jnp.exp(_lrelu(s0 + l2d_ref[...]) - m2_ref[...])
    tot = (q0_ref[:, :_L] + q1_ref[:, :_L] + w * s1)
    rr = lax.broadcasted_iota(jnp.int32, (_L, _L), 0)
    sel0 = (rr == 0).astype(jnp.float32)
    sel1 = (rr == 1).astype(jnp.float32)
    numb = jnp.dot(tot, sel0, preferred_element_type=jnp.float32)
    denb = jnp.dot(tot, sel1, preferred_element_type=jnp.float32)
    o = numb / (denb + 1e-16) + b2_ref[...]
    out_ref[...] = jnp.where(o > 0, o, jnp.exp(jnp.minimum(o, 0.0)) - 1.0)


def _tc(body, out_shape, *ins):
    return pl.pallas_call(body, out_shape=out_shape)(*ins)


def _make_edge_pass(n_nodes, n_edges, row_w, n_vregs_payload, ch):
    ew = n_edges // _NW
    nch = ew // ch
    assert ew % ch == 0 and ch % 8 == 0 and ch <= 128
    npad = -(-n_nodes // (128 * _NS)) * (128 * _NS)
    rpt = npad // _NS
    assert rpt % ch == 0
    mesh = plsc.VectorSubcoreMesh(core_axis_name="c", subcore_axis_name="s",
                                  num_cores=_NC, num_subcores=_NS)

    nb = 5
    assert nch % nb == 0
    ngrp = nch // nb

    @functools.partial(
        pl.kernel,
        out_type=jax.ShapeDtypeStruct((_NC, npad, row_w), jnp.float32),
        mesh=mesh,
        compiler_params=pltpu.CompilerParams(use_tc_tiling_on_sc=False),
        scratch_types=[
            pltpu.VMEM((nb, ch), jnp.int32),
            pltpu.VMEM((nb, ch), jnp.int32),
            pltpu.VMEM((nb, ch, row_w), jnp.float32),
            pltpu.VMEM((nb, ch, _L), jnp.float32),
            pltpu.VMEM((_L,), jnp.float32),
            pltpu.VMEM_SHARED((npad, row_w), jnp.float32),
            pltpu.SemaphoreType.DMA((nb,)),
            pltpu.SemaphoreType.DMA((nb,)),
            pltpu.SemaphoreType.DMA,
        ],
    )
    def k(src_hbm, dst_hbm, pack_hbm, adp_hbm, m_hbm, out_hbm,
          idxs_v, idxd_v, rows_v, ad_v, m_v, acc_sh,
          semi, semg, sems):
        cid = lax.axis_index("c")
        sid = lax.axis_index("s")

        def zrow(r, carry):
            for kk in range(row_w // _L):
                rows_v[0, r, pl.ds(kk * _L, _L)] = jnp.zeros((_L,), jnp.float32)
            return carry
        lax.fori_loop(0, ch, zrow, 0)
        dz = []
        for b in range(rpt // ch):
            dz.append(pltpu.async_copy(
                rows_v.at[0], acc_sh.at[pl.ds(sid * rpt + b * ch, ch)], sems))
        for d in dz:
            d.wait()
        pltpu.sync_copy(m_hbm, m_v)
        plsc.subcore_barrier()

        mvec = m_v[...]
        ebase = (cid * _NS + sid) * ew

        def group(g, carry):
            base0 = ebase + g * (nb * ch)
            di = []
            for b in range(nb):
                di.append(pltpu.async_copy(
                    src_hbm.at[pl.ds(base0 + b * ch, ch)],
                    idxs_v.at[b], semi.at[b]))
                di.append(pltpu.async_copy(
                    dst_hbm.at[pl.ds(base0 + b * ch, ch)],
                    idxd_v.at[b], semi.at[b]))
            dg = []
            for b in range(nb):
                di[2 * b].wait()
                di[2 * b + 1].wait()
                dg.append(pltpu.async_copy(
                    pack_hbm.at[idxs_v.at[b]], rows_v.at[b], semg.at[b]))
                dg.append(pltpu.async_copy(
                    adp_hbm.at[idxd_v.at[b]], ad_v.at[b], semg.at[b]))
            dsc = []
            for b in range(nb):
                dg[2 * b].wait()
                dg[2 * b + 1].wait()
                def edge(e2, c2, b=b):
                    asv = rows_v[b, e2, pl.ds(row_w - _L, _L)]
                    t = asv + ad_v[b, e2, pl.ds(0, _L)]
                    w = jnp.exp(jnp.maximum(t, _NEG * t) - mvec)
                    for kk in range(n_vregs_payload):
                        hv = rows_v[b, e2, pl.ds(kk * _L, _L)]
                        rows_v[b, e2, pl.ds(kk * _L, _L)] = hv * w
                    rows_v[b, e2, pl.ds(row_w - _L, _L)] = w
                    return c2
                lax.fori_loop(0, ch, edge, 0, unroll=4)
                dsc.append(pltpu.async_copy(
                    rows_v.at[b], acc_sh.at[idxd_v.at[b]], sems, add=True))
            for d in dsc:
                d.wait()
            return carry
        lax.fori_loop(0, ngrp, group, 0)

        plsc.subcore_barrier()
        pltpu.sync_copy(acc_sh.at[pl.ds(sid * rpt, rpt)],
                        out_hbm.at[cid, pl.ds(sid * rpt, rpt)])
    return k


def kernel(x, edge_index, W1, a1s, a1d, b1, W2, a2s, a2d, b2):
    n, f_in = x.shape
    e = edge_index.shape[1]
    h, f_hid = a1s.shape
    c1 = h * f_hid
    p1 = c1 + _L
    f32 = jnp.float32

    hh, ff = np.meshgrid(np.arange(h), np.arange(f_hid), indexing="ij")
    dest = ((ff // 2) * _L + (ff % 2) * h + hh).reshape(-1)
    inv = np.empty((c1,), np.int64)
    inv[dest] = np.arange(c1)

    w1flat = jnp.transpose(W1, (1, 0, 2)).reshape(f_in, c1)
    w1perm = w1flat[:, inv]
    avs = jnp.einsum("hif,hf->ih", W1, a1s)
    avd = jnp.einsum("hif,hf->ih", W1, a1d)
    wcat = jnp.concatenate(
        [w1perm, jnp.tile(avs, (1, 2)), jnp.tile(avd, (1, 2))], axis=1)

    b1p = b1.reshape(1, c1)[:, inv]
    w2p = W2[inv, :]
    w2rep = jnp.tile(w2p, (1, _L))
    wcat2 = jnp.concatenate(
        [w2rep, w2rep * a2s[0], w2rep * a2d[0]], axis=1)

    src = edge_index[0]
    dst = edge_index[1]

    hsrc, adp, m1 = _tc(
        _prep_body,
        (jax.ShapeDtypeStruct((n, p1), f32),
         jax.ShapeDtypeStruct((n, _L), f32),
         jax.ShapeDtypeStruct((1, _L), f32)),
        x, wcat)

    part1 = _make_edge_pass(n, e, p1, c1 // _L, 40)(
        src, dst, hsrc, adp, m1.reshape(_L))

    l2s, l2d, m2 = _tc(
        _mid_body,
        (jax.ShapeDtypeStruct((n, 2 * _L), f32),
         jax.ShapeDtypeStruct((n, _L), f32),
         jax.ShapeDtypeStruct((1, _L), f32)),
        part1[0, :n], part1[1, :n], hsrc, adp, m1, b1p, wcat2)

    part2 = _make_edge_pass(n, e, 2 * _L, 1, 80)(
        src, dst, l2s, l2d, m2.reshape(_L))

    out16 = _tc(
        _fin_body,
        jax.ShapeDtypeStruct((n, _L), f32),
        part2[0, :n], part2[1, :n], l2s, l2d, m2, b2.reshape(1, 1))

    return out16[:, :1]


# speedup vs baseline: 2.1489x; 1.5957x over previous
import functools

import numpy as np
import jax
import jax.numpy as jnp
from jax import lax
from jax.experimental import pallas as pl
from jax.experimental.pallas import tpu as pltpu
from jax.experimental.pallas import tpu_sc as plsc

_NEG = 0.2
_NC, _NS, _L = 2, 16, 16
_NW = _NC * _NS


def _lrelu(t):
    return jnp.maximum(t, _NEG * t)


def _prep_body(x_ref, wcat_ref, hsrc_ref, adp_ref, m1_ref):
    c1 = hsrc_ref.shape[1] - _L
    xw = jnp.dot(x_ref[...], wcat_ref[...], preferred_element_type=jnp.float32)
    asp = xw[:, c1:c1 + _L]
    adp = xw[:, c1 + _L:]
    hsrc_ref[:, :c1] = xw[:, :c1]
    hsrc_ref[:, c1:] = asp
    adp_ref[...] = adp
    m1_ref[...] = (jnp.max(asp, axis=0, keepdims=True)
                   + jnp.max(adp, axis=0, keepdims=True))


def _mid_body(p0_ref, p1_ref, hsrc_ref, adp_ref, m1_ref, b1p_ref, wcat2_ref,
              l2s_ref, l2d_ref, m2_ref):
    c1 = hsrc_ref.shape[1] - _L
    tot = p0_ref[...] + p1_ref[...]
    asp = hsrc_ref[:, c1:]
    wself = jnp.exp(_lrelu(asp + adp_ref[...]) - m1_ref[...])
    rr = lax.broadcasted_iota(jnp.int32, (_L, c1), 0)
    cc = lax.broadcasted_iota(jnp.int32, (_L, c1), 1)
    tile16 = (cc % _L == rr).astype(jnp.float32)
    wrep = jnp.dot(wself, tile16, preferred_element_type=jnp.float32)
    num = tot[:, :c1] + wrep * hsrc_ref[:, :c1]
    den = tot[:, c1:] + wself
    denrep = jnp.dot(den, tile16, preferred_element_type=jnp.float32)
    xh = num / (denrep + 1e-16) + b1p_ref[...]
    z = jnp.dot(xh, wcat2_ref[...], preferred_element_type=jnp.float32)
    h2 = z[:, :_L]
    as2 = z[:, _L:2 * _L]
    ad2 = z[:, 2 * _L:]
    m2_ref[...] = (jnp.max(as2, axis=0, keepdims=True)
                   + jnp.max(ad2, axis=0, keepdims=True))
    lanes = lax.broadcasted_iota(jnp.int32, (1, _L), 1)
    e0 = (lanes == 0).astype(jnp.float32)
    e1 = (lanes == 1).astype(jnp.float32)
    l2s_ref[:, :_L] = h2 * e0 + e1
    l2s_ref[:, _L:] = as2
    l2d_ref[...] = ad2


def _fin_body(q0_ref, q1_ref, l2s_ref, l2d_ref, m2_ref, b2_ref, out_ref):
    s1 = l2s_ref[:, :_L]
    s0 = l2s_ref[:, _L:]
    w = jnp.exp(_lrelu(s0 + l2d_ref[...]) - m2_ref[...])
    tot = (q0_ref[:, :_L] + q1_ref[:, :_L] + w * s1)
    rr = lax.broadcasted_iota(jnp.int32, (_L, _L), 0)
    sel0 = (rr == 0).astype(jnp.float32)
    sel1 = (rr == 1).astype(jnp.float32)
    numb = jnp.dot(tot, sel0, preferred_element_type=jnp.float32)
    denb = jnp.dot(tot, sel1, preferred_element_type=jnp.float32)
    o = numb / (denb + 1e-16) + b2_ref[...]
    out_ref[...] = jnp.where(o > 0, o, jnp.exp(jnp.minimum(o, 0.0)) - 1.0)


def _tc(body, out_shape, *ins):
    return pl.pallas_call(body, out_shape=out_shape)(*ins)


def _make_edge_pass(n_nodes, n_edges, row_w, n_vregs_payload, ch):
    ew = n_edges // _NW
    nch = ew // ch
    assert ew % ch == 0 and ch % 8 == 0 and ch <= 128
    npad = -(-n_nodes // (128 * _NS)) * (128 * _NS)
    rpt = npad // _NS
    assert rpt % ch == 0
    mesh = plsc.VectorSubcoreMesh(core_axis_name="c", subcore_axis_name="s",
                                  num_cores=_NC, num_subcores=_NS)

    nb = 5
    assert nch % nb == 0
    ngrp = nch // nb

    @functools.partial(
        pl.kernel,
        out_type=jax.ShapeDtypeStruct((_NC, npad, row_w), jnp.float32),
        mesh=mesh,
        compiler_params=pltpu.CompilerParams(use_tc_tiling_on_sc=False),
        scratch_types=[
            pltpu.VMEM((nb, ch), jnp.int32),
            pltpu.VMEM((nb, ch), jnp.int32),
            pltpu.VMEM((nb, ch, row_w), jnp.float32),
            pltpu.VMEM((nb, ch, _L), jnp.float32),
            pltpu.VMEM((_L,), jnp.float32),
            pltpu.VMEM_SHARED((npad, row_w), jnp.float32),
            pltpu.SemaphoreType.DMA((nb,)),
            pltpu.SemaphoreType.DMA((nb,)),
            pltpu.SemaphoreType.DMA,
        ],
    )
    def k(src_hbm, dst_hbm, pack_hbm, adp_hbm, m_hbm, out_hbm,
          idxs_v, idxd_v, rows_v, ad_v, m_v, acc_sh,
          semi, semg, sems):
        cid = lax.axis_index("c")
        sid = lax.axis_index("s")

        @plsc.parallel_loop(0, ch)
        def zrow(r):
            for kk in range(row_w // _L):
                rows_v[0, r, pl.ds(kk * _L, _L)] = jnp.zeros((_L,), jnp.float32)
        dz = []
        for b in range(rpt // ch):
            dz.append(pltpu.async_copy(
                rows_v.at[0], acc_sh.at[pl.ds(sid * rpt + b * ch, ch)], sems))
        for d in dz:
            d.wait()
        pltpu.sync_copy(m_hbm, m_v)
        plsc.subcore_barrier()

        mvec = m_v[...]
        ebase = (cid * _NS + sid) * ew

        def group(g, carry):
            base0 = ebase + g * (nb * ch)
            di = []
            for b in range(nb):
                di.append(pltpu.async_copy(
                    src_hbm.at[pl.ds(base0 + b * ch, ch)],
                    idxs_v.at[b], semi.at[b]))
                di.append(pltpu.async_copy(
                    dst_hbm.at[pl.ds(base0 + b * ch, ch)],
                    idxd_v.at[b], semi.at[b]))
            dg = []
            for b in range(nb):
                di[2 * b].wait()
                di[2 * b + 1].wait()
                dg.append(pltpu.async_copy(
                    pack_hbm.at[idxs_v.at[b]], rows_v.at[b], semg.at[b]))
                dg.append(pltpu.async_copy(
                    adp_hbm.at[idxd_v.at[b]], ad_v.at[b], semg.at[b]))
            dsc = []
            for b in range(nb):
                dg[2 * b].wait()
                dg[2 * b + 1].wait()
                @plsc.parallel_loop(0, ch, unroll=4)
                def edge(e2, b=b):
                    asv = rows_v[b, e2, pl.ds(row_w - _L, _L)]
                    t = asv + ad_v[b, e2, pl.ds(0, _L)]
                    w = jnp.exp(jnp.maximum(t, _NEG * t) - mvec)
                    for kk in range(n_vregs_payload):
                        hv = rows_v[b, e2, pl.ds(kk * _L, _L)]
                        rows_v[b, e2, pl.ds(kk * _L, _L)] = hv * w
                    rows_v[b, e2, pl.ds(row_w - _L, _L)] = w
                dsc.append(pltpu.async_copy(
                    rows_v.at[b], acc_sh.at[idxd_v.at[b]], sems, add=True))
            for d in dsc:
                d.wait()
            return carry
        lax.fori_loop(0, ngrp, group, 0)

        plsc.subcore_barrier()
        pltpu.sync_copy(acc_sh.at[pl.ds(sid * rpt, rpt)],
                        out_hbm.at[cid, pl.ds(sid * rpt, rpt)])
    return k


def kernel(x, edge_index, W1, a1s, a1d, b1, W2, a2s, a2d, b2):
    n, f_in = x.shape
    e = edge_index.shape[1]
    h, f_hid = a1s.shape
    c1 = h * f_hid
    p1 = c1 + _L
    f32 = jnp.float32

    hh, ff = np.meshgrid(np.arange(h), np.arange(f_hid), indexing="ij")
    dest = ((ff // 2) * _L + (ff % 2) * h + hh).reshape(-1)
    inv = np.empty((c1,), np.int64)
    inv[dest] = np.arange(c1)

    w1flat = jnp.transpose(W1, (1, 0, 2)).reshape(f_in, c1)
    w1perm = w1flat[:, inv]
    avs = jnp.einsum("hif,hf->ih", W1, a1s)
    avd = jnp.einsum("hif,hf->ih", W1, a1d)
    wcat = jnp.concatenate(
        [w1perm, jnp.tile(avs, (1, 2)), jnp.tile(avd, (1, 2))], axis=1)

    b1p = b1.reshape(1, c1)[:, inv]
    w2p = W2[inv, :]
    w2rep = jnp.tile(w2p, (1, _L))
    wcat2 = jnp.concatenate(
        [w2rep, w2rep * a2s[0], w2rep * a2d[0]], axis=1)

    src = edge_index[0]
    dst = edge_index[1]

    hsrc, adp, m1 = _tc(
        _prep_body,
        (jax.ShapeDtypeStruct((n, p1), f32),
         jax.ShapeDtypeStruct((n, _L), f32),
         jax.ShapeDtypeStruct((1, _L), f32)),
        x, wcat)

    part1 = _make_edge_pass(n, e, p1, c1 // _L, 40)(
        src, dst, hsrc, adp, m1.reshape(_L))

    l2s, l2d, m2 = _tc(
        _mid_body,
        (jax.ShapeDtypeStruct((n, 2 * _L), f32),
         jax.ShapeDtypeStruct((n, _L), f32),
         jax.ShapeDtypeStruct((1, _L), f32)),
        part1[0, :n], part1[1, :n], hsrc, adp, m1, b1p, wcat2)

    part2 = _make_edge_pass(n, e, 2 * _L, 1, 80)(
        src, dst, l2s, l2d, m2.reshape(_L))

    out16 = _tc(
        _fin_body,
        jax.ShapeDtypeStruct((n, _L), f32),
        part2[0, :n], part2[1, :n], l2s, l2d, m2, b2.reshape(1, 1))

    return out16[:, :1]


# speedup vs baseline: 2.1570x; 1.0038x over previous
import functools

import numpy as np
import jax
import jax.numpy as jnp
from jax import lax
from jax.experimental import pallas as pl
from jax.experimental.pallas import tpu as pltpu
from jax.experimental.pallas import tpu_sc as plsc

_NEG = 0.2
_NC, _NS, _L = 2, 16, 16
_NW = _NC * _NS


def _lrelu(t):
    return jnp.maximum(t, _NEG * t)


def _prep_body(x_ref, wcat_ref, hsrc_ref, adp_ref, m1_ref):
    c1 = hsrc_ref.shape[1] - _L
    xw = jnp.dot(x_ref[...], wcat_ref[...], preferred_element_type=jnp.float32)
    asp = xw[:, c1:c1 + _L]
    adp = xw[:, c1 + _L:]
    hsrc_ref[:, :c1] = xw[:, :c1]
    hsrc_ref[:, c1:] = asp
    adp_ref[...] = adp
    cur = jnp.concatenate([jnp.max(asp, axis=0, keepdims=True),
                           jnp.max(adp, axis=0, keepdims=True)], axis=0)
    i = pl.program_id(0)

    @pl.when(i == 0)
    def _():
        m1_ref[...] = cur

    @pl.when(i > 0)
    def _():
        m1_ref[...] = jnp.maximum(m1_ref[...], cur)


def _mid_body(p0_ref, p1_ref, hsrc_ref, adp_ref, m1_ref, b1p_ref, wcat2_ref,
              l2s_ref, l2d_ref, m2_ref):
    c1 = hsrc_ref.shape[1] - _L
    tot = p0_ref[...] + p1_ref[...]
    asp = hsrc_ref[:, c1:]
    wself = jnp.exp(_lrelu(asp + adp_ref[...]) - m1_ref[...])
    rr = lax.broadcasted_iota(jnp.int32, (_L, c1), 0)
    cc = lax.broadcasted_iota(jnp.int32, (_L, c1), 1)
    tile16 = (cc % _L == rr).astype(jnp.float32)
    wrep = jnp.dot(wself, tile16, preferred_element_type=jnp.float32)
    num = tot[:, :c1] + wrep * hsrc_ref[:, :c1]
    den = tot[:, c1:] + wself
    denrep = jnp.dot(den, tile16, preferred_element_type=jnp.float32)
    xh = num / (denrep + 1e-16) + b1p_ref[...]
    z = jnp.dot(xh, wcat2_ref[...], preferred_element_type=jnp.float32)
    h2 = z[:, :_L]
    as2 = z[:, _L:2 * _L]
    ad2 = z[:, 2 * _L:]
    cur = jnp.concatenate([jnp.max(as2, axis=0, keepdims=True),
                           jnp.max(ad2, axis=0, keepdims=True)], axis=0)
    i = pl.program_id(0)

    @pl.when(i == 0)
    def _():
        m2_ref[...] = cur

    @pl.when(i > 0)
    def _():
        m2_ref[...] = jnp.maximum(m2_ref[...], cur)
    lanes = lax.broadcasted_iota(jnp.int32, (1, _L), 1)
    e0 = (lanes == 0).astype(jnp.float32)
    e1 = (lanes == 1).astype(jnp.float32)
    l2s_ref[:, :_L] = h2 * e0 + e1
    l2s_ref[:, _L:] = as2
    l2d_ref[...] = ad2


def _fin_body(q0_ref, q1_ref, l2s_ref, l2d_ref, m2_ref, b2_ref, out_ref):
    s1 = l2s_ref[:, :_L]
    s0 = l2s_ref[:, _L:]
    w = jnp.exp(_lrelu(s0 + l2d_ref[...]) - m2_ref[...])
    tot = (q0_ref[:, :_L] + q1_ref[:, :_L] + w * s1)
    rr = lax.broadcasted_iota(jnp.int32, (_L, _L), 0)
    sel0 = (rr == 0).astype(jnp.float32)
    sel1 = (rr == 1).astype(jnp.float32)
    numb = jnp.dot(tot, sel0, preferred_element_type=jnp.float32)
    denb = jnp.dot(tot, sel1, preferred_element_type=jnp.float32)
    o = numb / (denb + 1e-16) + b2_ref[...]
    out_ref[...] = jnp.where(o > 0, o, jnp.exp(jnp.minimum(o, 0.0)) - 1.0)


_BLK = 2000


def _row_spec(w):
    return pl.BlockSpec((_BLK, w), lambda i: (i, 0))


def _bcast_spec(r, w):
    return pl.BlockSpec((r, w), lambda i: (0, 0))


def _make_edge_pass(n_nodes, n_edges, row_w, n_vregs_payload, ch):
    ew = n_edges // _NW
    nch = ew // ch
    assert ew % ch == 0 and ch % 8 == 0 and ch <= 128
    npad = -(-n_nodes // (128 * _NS)) * (128 * _NS)
    rpt = npad // _NS
    assert rpt % ch == 0
    mesh = plsc.VectorSubcoreMesh(core_axis_name="c", subcore_axis_name="s",
                                  num_cores=_NC, num_subcores=_NS)

    nb = 5
    assert nch % nb == 0
    ngrp = nch // nb

    @functools.partial(
        pl.kernel,
        out_type=jax.ShapeDtypeStruct((_NC, npad, row_w), jnp.float32),
        mesh=mesh,
        compiler_params=pltpu.CompilerParams(use_tc_tiling_on_sc=False),
        scratch_types=[
            pltpu.VMEM((nb, ch), jnp.int32),
            pltpu.VMEM((nb, ch), jnp.int32),
            pltpu.VMEM((nb, ch, row_w), jnp.float32),
            pltpu.VMEM((nb, ch, _L), jnp.float32),
            pltpu.VMEM((_L,), jnp.float32),
            pltpu.VMEM_SHARED((npad, row_w), jnp.float32),
            pltpu.SemaphoreType.DMA((nb,)),
            pltpu.SemaphoreType.DMA((nb,)),
            pltpu.SemaphoreType.DMA,
        ],
    )
    def k(src_hbm, dst_hbm, pack_hbm, adp_hbm, m_hbm, out_hbm,
          idxs_v, idxd_v, rows_v, ad_v, m_v, acc_sh,
          semi, semg, sems):
        cid = lax.axis_index("c")
        sid = lax.axis_index("s")

        @plsc.parallel_loop(0, ch)
        def zrow(r):
            for kk in range(row_w // _L):
                rows_v[0, r, pl.ds(kk * _L, _L)] = jnp.zeros((_L,), jnp.float32)
        dz = []
        for b in range(rpt // ch):
            dz.append(pltpu.async_copy(
                rows_v.at[0], acc_sh.at[pl.ds(sid * rpt + b * ch, ch)], sems))
        for d in dz:
            d.wait()
        pltpu.sync_copy(m_hbm, m_v)
        plsc.subcore_barrier()

        mvec = m_v[...]
        ebase = (cid * _NS + sid) * ew

        def group(g, carry):
            base0 = ebase + g * (nb * ch)
            di = []
            for b in range(nb):
                di.append(pltpu.async_copy(
                    src_hbm.at[pl.ds(base0 + b * ch, ch)],
                    idxs_v.at[b], semi.at[b]))
                di.append(pltpu.async_copy(
                    dst_hbm.at[pl.ds(base0 + b * ch, ch)],
                    idxd_v.at[b], semi.at[b]))
            dg = []
            for b in range(nb):
                di[2 * b].wait()
                di[2 * b + 1].wait()
                dg.append(pltpu.async_copy(
                    pack_hbm.at[idxs_v.at[b]], rows_v.at[b], semg.at[b]))
                dg.append(pltpu.async_copy(
                    adp_hbm.at[idxd_v.at[b]], ad_v.at[b], semg.at[b]))
            dsc = []
            for b in range(nb):
                dg[2 * b].wait()
                dg[2 * b + 1].wait()
                @plsc.parallel_loop(0, ch, unroll=4)
                def edge(e2, b=b):
                    asv = rows_v[b, e2, pl.ds(row_w - _L, _L)]
                    t = asv + ad_v[b, e2, pl.ds(0, _L)]
                    w = jnp.exp(jnp.maximum(t, _NEG * t) - mvec)
                    for kk in range(n_vregs_payload):
                        hv = rows_v[b, e2, pl.ds(kk * _L, _L)]
                        rows_v[b, e2, pl.ds(kk * _L, _L)] = hv * w
                    rows_v[b, e2, pl.ds(row_w - _L, _L)] = w
                dsc.append(pltpu.async_copy(
                    rows_v.at[b], acc_sh.at[idxd_v.at[b]], sems, add=True))
            for d in dsc:
                d.wait()
            return carry
        lax.fori_loop(0, ngrp, group, 0)

        plsc.subcore_barrier()
        pltpu.sync_copy(acc_sh.at[pl.ds(sid * rpt, rpt)],
                        out_hbm.at[cid, pl.ds(sid * rpt, rpt)])
    return k


def kernel(x, edge_index, W1, a1s, a1d, b1, W2, a2s, a2d, b2):
    n, f_in = x.shape
    e = edge_index.shape[1]
    h, f_hid = a1s.shape
    c1 = h * f_hid
    p1 = c1 + _L
    f32 = jnp.float32

    hh, ff = np.meshgrid(np.arange(h), np.arange(f_hid), indexing="ij")
    dest = ((ff // 2) * _L + (ff % 2) * h + hh).reshape(-1)
    inv = np.empty((c1,), np.int64)
    inv[dest] = np.arange(c1)

    w1flat = jnp.transpose(W1, (1, 0, 2)).reshape(f_in, c1)
    w1perm = w1flat[:, inv]
    avs = jnp.einsum("hif,hf->ih", W1, a1s)
    avd = jnp.einsum("hif,hf->ih", W1, a1d)
    wcat = jnp.concatenate(
        [w1perm, jnp.tile(avs, (1, 2)), jnp.tile(avd, (1, 2))], axis=1)

    b1p = b1.reshape(1, c1)[:, inv]
    w2p = W2[inv, :]
    w2rep = jnp.tile(w2p, (1, _L))
    wcat2 = jnp.concatenate(
        [w2rep, w2rep * a2s[0], w2rep * a2d[0]], axis=1)

    src = edge_index[0]
    dst = edge_index[1]

    grid = (n // _BLK,)
    assert n % _BLK == 0

    hsrc, adp, m1 = pl.pallas_call(
        _prep_body,
        grid=grid,
        in_specs=[_row_spec(f_in), _bcast_spec(f_in, c1 + 2 * _L)],
        out_specs=[_row_spec(p1), _row_spec(_L), _bcast_spec(2, _L)],
        out_shape=(jax.ShapeDtypeStruct((n, p1), f32),
                   jax.ShapeDtypeStruct((n, _L), f32),
                   jax.ShapeDtypeStruct((2, _L), f32)),
    )(x, wcat)
    m1c = m1[0:1] + m1[1:2]

    part1 = _make_edge_pass(n, e, p1, c1 // _L, 40)(
        src, dst, hsrc, adp, m1c.reshape(_L))

    l2s, l2d, m2 = pl.pallas_call(
        _mid_body,
        grid=grid,
        in_specs=[_row_spec(p1), _row_spec(p1), _row_spec(p1), _row_spec(_L),
                  _bcast_spec(1, _L), _bcast_spec(1, c1),
                  _bcast_spec(c1, 3 * _L)],
        out_specs=[_row_spec(2 * _L), _row_spec(_L), _bcast_spec(2, _L)],
        out_shape=(jax.ShapeDtypeStruct((n, 2 * _L), f32),
                   jax.ShapeDtypeStruct((n, _L), f32),
                   jax.ShapeDtypeStruct((2, _L), f32)),
    )(part1[0, :n], part1[1, :n], hsrc, adp, m1c, b1p, wcat2)
    m2c = m2[0:1] + m2[1:2]

    part2 = _make_edge_pass(n, e, 2 * _L, 1, 80)(
        src, dst, l2s, l2d, m2c.reshape(_L))

    out16 = pl.pallas_call(
        _fin_body,
        grid=grid,
        in_specs=[_row_spec(2 * _L), _row_spec(2 * _L), _row_spec(2 * _L),
                  _row_spec(_L), _bcast_spec(1, _L), _bcast_spec(1, 1)],
        out_specs=[_row_spec(_L)],
        out_shape=[jax.ShapeDtypeStruct((n, _L), f32)],
    )(part2[0, :n], part2[1, :n], l2s, l2d, m2c, b2.reshape(1, 1))[0]

    return out16[:, :1]


# speedup vs baseline: 2.1769x; 1.0092x over previous
import functools

import numpy as np
import jax
import jax.numpy as jnp
from jax import lax
from jax.experimental import pallas as pl
from jax.experimental.pallas import tpu as pltpu
from jax.experimental.pallas import tpu_sc as plsc

_NEG = 0.2
_NC, _NS, _L = 2, 16, 16
_NW = _NC * _NS


def _lrelu(t):
    return jnp.maximum(t, _NEG * t)


def _prep_body(x_ref, wcat_ref, hsrc_ref, hb_ref, asp_ref, adp_ref, m1_ref):
    c1 = hsrc_ref.shape[1] - _L
    xw = jnp.dot(x_ref[...], wcat_ref[...], preferred_element_type=jnp.float32)
    asp = xw[:, 2 * c1:2 * c1 + _L]
    adp = xw[:, 2 * c1 + _L:]
    hsrc_ref[:, :c1] = xw[:, :c1]
    hsrc_ref[:, c1:] = asp
    hb_ref[...] = xw[:, c1:2 * c1].astype(jnp.bfloat16)
    asp_ref[...] = asp
    adp_ref[...] = adp
    cur = jnp.concatenate([jnp.max(asp, axis=0, keepdims=True),
                           jnp.max(adp, axis=0, keepdims=True)], axis=0)
    i = pl.program_id(0)

    @pl.when(i == 0)
    def _():
        m1_ref[...] = cur

    @pl.when(i > 0)
    def _():
        m1_ref[...] = jnp.maximum(m1_ref[...], cur)


def _mid_body(p0_ref, p1_ref, hsrc_ref, adp_ref, m1_ref, b1p_ref, wcat2_ref,
              l2s_ref, l2d_ref, m2_ref):
    c1 = hsrc_ref.shape[1] - _L
    tot = p0_ref[...] + p1_ref[...]
    asp = hsrc_ref[:, c1:]
    wself = jnp.exp(_lrelu(asp + adp_ref[...]) - m1_ref[...])
    rr = lax.broadcasted_iota(jnp.int32, (_L, c1), 0)
    cc = lax.broadcasted_iota(jnp.int32, (_L, c1), 1)
    tile16 = (cc % _L == rr).astype(jnp.float32)
    wrep = jnp.dot(wself, tile16, preferred_element_type=jnp.float32)
    num = tot[:, :c1] + wrep * hsrc_ref[:, :c1]
    den = tot[:, c1:] + wself
    denrep = jnp.dot(den, tile16, preferred_element_type=jnp.float32)
    xh = num / (denrep + 1e-16) + b1p_ref[...]
    z = jnp.dot(xh, wcat2_ref[...], preferred_element_type=jnp.float32)
    h2 = z[:, :_L]
    as2 = z[:, _L:2 * _L]
    ad2 = z[:, 2 * _L:]
    cur = jnp.concatenate([jnp.max(as2, axis=0, keepdims=True),
                           jnp.max(ad2, axis=0, keepdims=True)], axis=0)
    i = pl.program_id(0)

    @pl.when(i == 0)
    def _():
        m2_ref[...] = cur

    @pl.when(i > 0)
    def _():
        m2_ref[...] = jnp.maximum(m2_ref[...], cur)
    lanes = lax.broadcasted_iota(jnp.int32, (1, _L), 1)
    e0 = (lanes == 0).astype(jnp.float32)
    e1 = (lanes == 1).astype(jnp.float32)
    l2s_ref[:, :_L] = h2 * e0 + e1
    l2s_ref[:, _L:] = as2
    l2d_ref[...] = ad2


def _fin_body(q0_ref, q1_ref, l2s_ref, l2d_ref, m2_ref, b2_ref, out_ref):
    s1 = l2s_ref[:, :_L]
    s0 = l2s_ref[:, _L:]
    w = jnp.exp(_lrelu(s0 + l2d_ref[...]) - m2_ref[...])
    tot = (q0_ref[:, :_L] + q1_ref[:, :_L] + w * s1)
    rr = lax.broadcasted_iota(jnp.int32, (_L, _L), 0)
    sel0 = (rr == 0).astype(jnp.float32)
    sel1 = (rr == 1).astype(jnp.float32)
    numb = jnp.dot(tot, sel0, preferred_element_type=jnp.float32)
    denb = jnp.dot(tot, sel1, preferred_element_type=jnp.float32)
    o = numb / (denb + 1e-16) + b2_ref[...]
    out_ref[...] = jnp.where(o > 0, o, jnp.exp(jnp.minimum(o, 0.0)) - 1.0)


_BLK = 2000


def _row_spec(w):
    return pl.BlockSpec((_BLK, w), lambda i: (i, 0))


def _bcast_spec(r, w):
    return pl.BlockSpec((r, w), lambda i: (0, 0))


def _make_edge_pass_bf16(n_nodes, n_edges, row_w, ch):
    c1 = row_w - _L
    ew = n_edges // _NW
    nch = ew // ch
    assert ew % ch == 0 and ch % 8 == 0 and ch <= 128
    npad = -(-n_nodes // (128 * _NS)) * (128 * _NS)
    rpt = npad // _NS
    assert rpt % ch == 0
    mesh = plsc.VectorSubcoreMesh(core_axis_name="c", subcore_axis_name="s",
                                  num_cores=_NC, num_subcores=_NS)

    nb = 5
    ns = 2
    assert nch % nb == 0
    ngrp = nch // nb

    @functools.partial(
        pl.kernel,
        out_type=jax.ShapeDtypeStruct((_NC, npad, row_w), jnp.float32),
        mesh=mesh,
        compiler_params=pltpu.CompilerParams(use_tc_tiling_on_sc=False),
        scratch_types=[
            pltpu.VMEM((nb, ch), jnp.int32),
            pltpu.VMEM((nb, ch), jnp.int32),
            pltpu.VMEM((nb, ch, c1 // 2), jnp.int32),
            pltpu.VMEM((nb, ch, _L), jnp.float32),
            pltpu.VMEM((nb, ch, _L), jnp.float32),
            pltpu.VMEM((ns, ch, row_w), jnp.float32),
            pltpu.VMEM((_L,), jnp.float32),
            pltpu.VMEM_SHARED((npad, row_w), jnp.float32),
            pltpu.SemaphoreType.DMA((nb,)),
            pltpu.SemaphoreType.DMA((nb,)),
            pltpu.SemaphoreType.DMA((ns,)),
        ],
    )
    def k(src_hbm, dst_hbm, hb_hbm, as_hbm, ad_hbm, m_hbm, out_hbm,
          idxs_v, idxd_v, hb_v, as_v, ad_v, sc_v, m_v, acc_sh,
          semi, semg, sems):
        cid = lax.axis_index("c")
        sid = lax.axis_index("s")

        @plsc.parallel_loop(0, ch)
        def zrow(r):
            for kk in range(row_w // _L):
                sc_v[0, r, pl.ds(kk * _L, _L)] = jnp.zeros((_L,), jnp.float32)
        dz = []
        for b in range(rpt // ch):
            dz.append(pltpu.async_copy(
                sc_v.at[0], acc_sh.at[pl.ds(sid * rpt + b * ch, ch)],
                sems.at[0]))
        for d in dz:
            d.wait()
        pltpu.sync_copy(m_hbm, m_v)
        plsc.subcore_barrier()

        mvec = m_v[...]
        ebase = (cid * _NS + sid) * ew

        def group(g, carry):
            base0 = ebase + g * (nb * ch)
            di = []
            for b in range(nb):
                di.append(pltpu.async_copy(
                    src_hbm.at[pl.ds(base0 + b * ch, ch)],
                    idxs_v.at[b], semi.at[b]))
                di.append(pltpu.async_copy(
                    dst_hbm.at[pl.ds(base0 + b * ch, ch)],
                    idxd_v.at[b], semi.at[b]))
            dg = []
            for b in range(nb):
                di[2 * b].wait()
                di[2 * b + 1].wait()
                dg.append(pltpu.async_copy(
                    hb_hbm.at[idxs_v.at[b]], hb_v.at[b], semg.at[b]))
                dg.append(pltpu.async_copy(
                    as_hbm.at[idxs_v.at[b]], as_v.at[b], semg.at[b]))
                dg.append(pltpu.async_copy(
                    ad_hbm.at[idxd_v.at[b]], ad_v.at[b], semg.at[b]))
            dsc = []
            for b in range(nb):
                sb = b % ns
                if b >= ns:
                    dsc[b - ns].wait()
                dg[3 * b].wait()
                dg[3 * b + 1].wait()
                dg[3 * b + 2].wait()

                @plsc.parallel_loop(0, ch, unroll=4)
                def edge(e2, b=b, sb=sb):
                    asv = as_v[b, e2, pl.ds(0, _L)]
                    t = asv + ad_v[b, e2, pl.ds(0, _L)]
                    w = jnp.exp(jnp.maximum(t, _NEG * t) - mvec)
                    for p in range(c1 // (2 * _L)):
                        hw = hb_v[b, e2, pl.ds(_L * p, _L)]
                        ha = lax.bitcast_convert_type(hw << 16, jnp.float32)
                        hb2 = lax.bitcast_convert_type(
                            hw & jnp.int32(-65536), jnp.float32)
                        sc_v[sb, e2, pl.ds(_L * 2 * p, _L)] = ha * w
                        sc_v[sb, e2, pl.ds(_L * (2 * p + 1), _L)] = hb2 * w
                    sc_v[sb, e2, pl.ds(row_w - _L, _L)] = w
                dsc.append(pltpu.async_copy(
                    sc_v.at[sb], acc_sh.at[idxd_v.at[b]], sems.at[sb],
                    add=True))
            for d in dsc[nb - ns:]:
                d.wait()
            return carry
        lax.fori_loop(0, ngrp, group, 0)

        plsc.subcore_barrier()
        pltpu.sync_copy(acc_sh.at[pl.ds(sid * rpt, rpt)],
                        out_hbm.at[cid, pl.ds(sid * rpt, rpt)])
    return k


def _make_edge_pass(n_nodes, n_edges, row_w, n_vregs_payload, ch):
    ew = n_edges // _NW
    nch = ew // ch
    assert ew % ch == 0 and ch % 8 == 0 and ch <= 128
    npad = -(-n_nodes // (128 * _NS)) * (128 * _NS)
    rpt = npad // _NS
    assert rpt % ch == 0
    mesh = plsc.VectorSubcoreMesh(core_axis_name="c", subcore_axis_name="s",
                                  num_cores=_NC, num_subcores=_NS)

    nb = 5
    assert nch % nb == 0
    ngrp = nch // nb

    @functools.partial(
        pl.kernel,
        out_type=jax.ShapeDtypeStruct((_NC, npad, row_w), jnp.float32),
        mesh=mesh,
        compiler_params=pltpu.CompilerParams(use_tc_tiling_on_sc=False),
        scratch_types=[
            pltpu.VMEM((nb, ch), jnp.int32),
            pltpu.VMEM((nb, ch), jnp.int32),
            pltpu.VMEM((nb, ch, row_w), jnp.float32),
            pltpu.VMEM((nb, ch, _L), jnp.float32),
            pltpu.VMEM((_L,), jnp.float32),
            pltpu.VMEM_SHARED((npad, row_w), jnp.float32),
            pltpu.SemaphoreType.DMA((nb,)),
            pltpu.SemaphoreType.DMA((nb,)),
            pltpu.SemaphoreType.DMA,
        ],
    )
    def k(src_hbm, dst_hbm, pack_hbm, adp_hbm, m_hbm, out_hbm,
          idxs_v, idxd_v, rows_v, ad_v, m_v, acc_sh,
          semi, semg, sems):
        cid = lax.axis_index("c")
        sid = lax.axis_index("s")

        @plsc.parallel_loop(0, ch)
        def zrow(r):
            for kk in range(row_w // _L):
                rows_v[0, r, pl.ds(kk * _L, _L)] = jnp.zeros((_L,), jnp.float32)
        dz = []
        for b in range(rpt // ch):
            dz.append(pltpu.async_copy(
                rows_v.at[0], acc_sh.at[pl.ds(sid * rpt + b * ch, ch)], sems))
        for d in dz:
            d.wait()
        pltpu.sync_copy(m_hbm, m_v)
        plsc.subcore_barrier()

        mvec = m_v[...]
        ebase = (cid * _NS + sid) * ew

        def group(g, carry):
            base0 = ebase + g * (nb * ch)
            di = []
            for b in range(nb):
                di.append(pltpu.async_copy(
                    src_hbm.at[pl.ds(base0 + b * ch, ch)],
                    idxs_v.at[b], semi.at[b]))
                di.append(pltpu.async_copy(
                    dst_hbm.at[pl.ds(base0 + b * ch, ch)],
                    idxd_v.at[b], semi.at[b]))
            dg = []
            for b in range(nb):
                di[2 * b].wait()
                di[2 * b + 1].wait()
                dg.append(pltpu.async_copy(
                    pack_hbm.at[idxs_v.at[b]], rows_v.at[b], semg.at[b]))
                dg.append(pltpu.async_copy(
                    adp_hbm.at[idxd_v.at[b]], ad_v.at[b], semg.at[b]))
            dsc = []
            for b in range(nb):
                dg[2 * b].wait()
                dg[2 * b + 1].wait()
                @plsc.parallel_loop(0, ch, unroll=4)
                def edge(e2, b=b):
                    asv = rows_v[b, e2, pl.ds(row_w - _L, _L)]
                    t = asv + ad_v[b, e2, pl.ds(0, _L)]
                    w = jnp.exp(jnp.maximum(t, _NEG * t) - mvec)
                    for kk in range(n_vregs_payload):
                        hv = rows_v[b, e2, pl.ds(kk * _L, _L)]
                        rows_v[b, e2, pl.ds(kk * _L, _L)] = hv * w
                    rows_v[b, e2, pl.ds(row_w - _L, _L)] = w
                dsc.append(pltpu.async_copy(
                    rows_v.at[b], acc_sh.at[idxd_v.at[b]], sems, add=True))
            for d in dsc:
                d.wait()
            return carry
        lax.fori_loop(0, ngrp, group, 0)

        plsc.subcore_barrier()
        pltpu.sync_copy(acc_sh.at[pl.ds(sid * rpt, rpt)],
                        out_hbm.at[cid, pl.ds(sid * rpt, rpt)])
    return k


def kernel(x, edge_index, W1, a1s, a1d, b1, W2, a2s, a2d, b2):
    n, f_in = x.shape
    e = edge_index.shape[1]
    h, f_hid = a1s.shape
    c1 = h * f_hid
    p1 = c1 + _L
    f32 = jnp.float32

    hh, ff = np.meshgrid(np.arange(h), np.arange(f_hid), indexing="ij")
    dest = ((ff // 2) * _L + (ff % 2) * h + hh).reshape(-1)
    inv = np.empty((c1,), np.int64)
    inv[dest] = np.arange(c1)

    cc_ = np.arange(c1)
    pp_, rr_ = cc_ // (2 * _L), cc_ % (2 * _L)
    tobf = np.where(rr_ < _L, 2 * _L * pp_ + 2 * rr_,
                    2 * _L * pp_ + 2 * (rr_ - _L) + 1)
    invb = np.empty((c1,), np.int64)
    invb[tobf[dest]] = np.arange(c1)

    w1flat = jnp.transpose(W1, (1, 0, 2)).reshape(f_in, c1)
    w1perm = w1flat[:, inv]
    w1permb = w1flat[:, invb]
    avs = jnp.einsum("hif,hf->ih", W1, a1s)
    avd = jnp.einsum("hif,hf->ih", W1, a1d)
    wcat = jnp.concatenate(
        [w1perm, w1permb,
         jnp.tile(avs, (1, 2)), jnp.tile(avd, (1, 2))], axis=1)

    b1p = b1.reshape(1, c1)[:, inv]
    w2p = W2[inv, :]
    w2rep = jnp.tile(w2p, (1, _L))
    wcat2 = jnp.concatenate(
        [w2rep, w2rep * a2s[0], w2rep * a2d[0]], axis=1)

    src = edge_index[0]
    dst = edge_index[1]

    grid = (n // _BLK,)
    assert n % _BLK == 0

    hsrc, hb, asp, adp, m1 = pl.pallas_call(
        _prep_body,
        grid=grid,
        in_specs=[_row_spec(f_in), _bcast_spec(f_in, 2 * c1 + 2 * _L)],
        out_specs=[_row_spec(p1), _row_spec(c1), _row_spec(_L),
                   _row_spec(_L), _bcast_spec(2, _L)],
        out_shape=(jax.ShapeDtypeStruct((n, p1), f32),
                   jax.ShapeDtypeStruct((n, c1), jnp.bfloat16),
                   jax.ShapeDtypeStruct((n, _L), f32),
                   jax.ShapeDtypeStruct((n, _L), f32),
                   jax.ShapeDtypeStruct((2, _L), f32)),
    )(x, wcat)
    m1c = m1[0:1] + m1[1:2]
    hb32 = lax.bitcast_convert_type(hb.reshape(n, c1 // 2, 2), jnp.int32)

    part1 = _make_edge_pass_bf16(n, e, p1, 40)(
        src, dst, hb32, asp, adp, m1c.reshape(_L))

    l2s, l2d, m2 = pl.pallas_call(
        _mid_body,
        grid=grid,
        in_specs=[_row_spec(p1), _row_spec(p1), _row_spec(p1), _row_spec(_L),
                  _bcast_spec(1, _L), _bcast_spec(1, c1),
                  _bcast_spec(c1, 3 * _L)],
        out_specs=[_row_spec(2 * _L), _row_spec(_L), _bcast_spec(2, _L)],
        out_shape=(jax.ShapeDtypeStruct((n, 2 * _L), f32),
                   jax.ShapeDtypeStruct((n, _L), f32),
                   jax.ShapeDtypeStruct((2, _L), f32)),
    )(part1[0, :n], part1[1, :n], hsrc, adp, m1c, b1p, wcat2)
    m2c = m2[0:1] + m2[1:2]

    part2 = _make_edge_pass(n, e, 2 * _L, 1, 80)(
        src, dst, l2s, l2d, m2c.reshape(_L))

    out16 = pl.pallas_call(
        _fin_body,
        grid=grid,
        in_specs=[_row_spec(2 * _L), _row_spec(2 * _L), _row_spec(2 * _L),
                  _row_spec(_L), _bcast_spec(1, _L), _bcast_spec(1, 1)],
        out_specs=[_row_spec(_L)],
        out_shape=[jax.ShapeDtypeStruct((n, _L), f32)],
    )(part2[0, :n], part2[1, :n], l2s, l2d, m2c, b2.reshape(1, 1))[0]

    return out16[:, :1]


# speedup vs baseline: 2.1981x; 1.0098x over previous
import functools

import numpy as np
import jax
import jax.numpy as jnp
from jax import lax
from jax.experimental import pallas as pl
from jax.experimental.pallas import tpu as pltpu
from jax.experimental.pallas import tpu_sc as plsc

_NEG = 0.2
_NC, _NS, _L = 2, 16, 16
_NW = _NC * _NS


def _lrelu(t):
    return jnp.maximum(t, _NEG * t)


def _prep_body(x_ref, wcat_ref, hsrc_ref, hb_ref, asp_ref, adp_ref, m1_ref):
    c1 = hsrc_ref.shape[1] - _L
    xw = jnp.dot(x_ref[...], wcat_ref[...], preferred_element_type=jnp.float32)
    asp = xw[:, 2 * c1:2 * c1 + _L]
    adp = xw[:, 2 * c1 + _L:]
    hsrc_ref[:, :c1] = xw[:, :c1]
    hsrc_ref[:, c1:] = asp
    hb_ref[...] = xw[:, c1:2 * c1].astype(jnp.bfloat16)
    asp_ref[...] = asp
    adp_ref[...] = adp
    cur = jnp.concatenate([jnp.max(asp, axis=0, keepdims=True),
                           jnp.max(adp, axis=0, keepdims=True)], axis=0)
    i = pl.program_id(0)

    @pl.when(i == 0)
    def _():
        m1_ref[...] = cur

    @pl.when(i > 0)
    def _():
        m1_ref[...] = jnp.maximum(m1_ref[...], cur)


def _mid_body(p0_ref, p1_ref, hsrc_ref, adp_ref, m1_ref, b1p_ref, wcat2_ref,
              l2s_ref, l2d_ref, m2_ref):
    c1 = hsrc_ref.shape[1] - _L
    tot = p0_ref[...] + p1_ref[...]
    asp = hsrc_ref[:, c1:]
    wself = jnp.exp(_lrelu(asp + adp_ref[...]) - m1_ref[...])
    rr = lax.broadcasted_iota(jnp.int32, (_L, c1), 0)
    cc = lax.broadcasted_iota(jnp.int32, (_L, c1), 1)
    tile16 = (cc % _L == rr).astype(jnp.float32)
    wrep = jnp.dot(wself, tile16, preferred_element_type=jnp.float32)
    num = tot[:, :c1] + wrep * hsrc_ref[:, :c1]
    den = tot[:, c1:] + wself
    denrep = jnp.dot(den, tile16, preferred_element_type=jnp.float32)
    xh = num / (denrep + 1e-16) + b1p_ref[...]
    z = jnp.dot(xh, wcat2_ref[...], preferred_element_type=jnp.float32)
    h2 = z[:, :_L]
    as2 = z[:, _L:2 * _L]
    ad2 = z[:, 2 * _L:]
    cur = jnp.concatenate([jnp.max(as2, axis=0, keepdims=True),
                           jnp.max(ad2, axis=0, keepdims=True)], axis=0)
    i = pl.program_id(0)

    @pl.when(i == 0)
    def _():
        m2_ref[...] = cur

    @pl.when(i > 0)
    def _():
        m2_ref[...] = jnp.maximum(m2_ref[...], cur)
    lanes = lax.broadcasted_iota(jnp.int32, (1, _L), 1)
    e0 = (lanes == 0).astype(jnp.float32)
    e1 = (lanes == 1).astype(jnp.float32)
    l2s_ref[:, :_L] = h2 * e0 + e1
    l2s_ref[:, _L:] = as2
    l2d_ref[...] = ad2


def _fin_body(q0_ref, q1_ref, l2s_ref, l2d_ref, m2_ref, b2_ref, out_ref):
    s1 = l2s_ref[:, :_L]
    s0 = l2s_ref[:, _L:]
    w = jnp.exp(_lrelu(s0 + l2d_ref[...]) - m2_ref[...])
    tot = (q0_ref[:, :_L] + q1_ref[:, :_L] + w * s1)
    rr = lax.broadcasted_iota(jnp.int32, (_L, _L), 0)
    sel0 = (rr == 0).astype(jnp.float32)
    sel1 = (rr == 1).astype(jnp.float32)
    numb = jnp.dot(tot, sel0, preferred_element_type=jnp.float32)
    denb = jnp.dot(tot, sel1, preferred_element_type=jnp.float32)
    o = numb / (denb + 1e-16) + b2_ref[...]
    out_ref[...] = jnp.where(o > 0, o, jnp.exp(jnp.minimum(o, 0.0)) - 1.0)


_BLK = 2000


def _row_spec(w):
    return pl.BlockSpec((_BLK, w), lambda i: (i, 0))


def _bcast_spec(r, w):
    return pl.BlockSpec((r, w), lambda i: (0, 0))


def _make_edge_pass_bf16(n_nodes, n_edges, row_w, ch):
    c1 = row_w - _L
    ew = n_edges // _NW
    nch = ew // ch
    assert ew % ch == 0 and ch % 8 == 0 and ch <= 128
    npad = -(-n_nodes // (128 * _NS)) * (128 * _NS)
    rpt = npad // _NS
    assert rpt % ch == 0
    mesh = plsc.VectorSubcoreMesh(core_axis_name="c", subcore_axis_name="s",
                                  num_cores=_NC, num_subcores=_NS)

    nb = 5
    ns = 3
    assert nch % nb == 0
    ngrp = nch // nb

    @functools.partial(
        pl.kernel,
        out_type=jax.ShapeDtypeStruct((_NC, npad, row_w), jnp.float32),
        mesh=mesh,
        compiler_params=pltpu.CompilerParams(use_tc_tiling_on_sc=False),
        scratch_types=[
            pltpu.VMEM((nb, ch), jnp.int32),
            pltpu.VMEM((nb, ch), jnp.int32),
            pltpu.VMEM((nb, ch, c1 // 2), jnp.int32),
            pltpu.VMEM((nb, ch, _L), jnp.float32),
            pltpu.VMEM((nb, ch, _L), jnp.float32),
            pltpu.VMEM((ns, ch, row_w), jnp.float32),
            pltpu.VMEM((_L,), jnp.float32),
            pltpu.VMEM_SHARED((npad, row_w), jnp.float32),
            pltpu.SemaphoreType.DMA((nb,)),
            pltpu.SemaphoreType.DMA((nb,)),
            pltpu.SemaphoreType.DMA((ns,)),
        ],
    )
    def k(src_hbm, dst_hbm, hb_hbm, as_hbm, ad_hbm, m_hbm, out_hbm,
          idxs_v, idxd_v, hb_v, as_v, ad_v, sc_v, m_v, acc_sh,
          semi, semg, sems):
        cid = lax.axis_index("c")
        sid = lax.axis_index("s")

        @plsc.parallel_loop(0, ch)
        def zrow(r):
            for kk in range(row_w // _L):
                sc_v[0, r, pl.ds(kk * _L, _L)] = jnp.zeros((_L,), jnp.float32)
        dz = []
        for b in range(rpt // ch):
            dz.append(pltpu.async_copy(
                sc_v.at[0], acc_sh.at[pl.ds(sid * rpt + b * ch, ch)],
                sems.at[0]))
        for d in dz:
            d.wait()
        pltpu.sync_copy(m_hbm, m_v)
        plsc.subcore_barrier()

        mvec = m_v[...]
        ebase = (cid * _NS + sid) * ew

        def group(g, carry):
            base0 = ebase + g * (nb * ch)
            di = []
            for b in range(nb):
                di.append(pltpu.async_copy(
                    src_hbm.at[pl.ds(base0 + b * ch, ch)],
                    idxs_v.at[b], semi.at[b]))
                di.append(pltpu.async_copy(
                    dst_hbm.at[pl.ds(base0 + b * ch, ch)],
                    idxd_v.at[b], semi.at[b]))
            dg = []
            for b in range(nb):
                di[2 * b].wait()
                di[2 * b + 1].wait()
                dg.append(pltpu.async_copy(
                    hb_hbm.at[idxs_v.at[b]], hb_v.at[b], semg.at[b]))
                dg.append(pltpu.async_copy(
                    as_hbm.at[idxs_v.at[b]], as_v.at[b], semg.at[b]))
                dg.append(pltpu.async_copy(
                    ad_hbm.at[idxd_v.at[b]], ad_v.at[b], semg.at[b]))
            dsc = []
            for b in range(nb):
                sb = b % ns
                if b >= ns:
                    dsc[b - ns].wait()
                dg[3 * b].wait()
                dg[3 * b + 1].wait()
                dg[3 * b + 2].wait()

                @plsc.parallel_loop(0, ch, unroll=4)
                def edge(e2, b=b, sb=sb):
                    asv = as_v[b, e2, pl.ds(0, _L)]
                    t = asv + ad_v[b, e2, pl.ds(0, _L)]
                    w = jnp.exp(jnp.maximum(t, _NEG * t) - mvec)
                    for p in range(c1 // (2 * _L)):
                        hw = hb_v[b, e2, pl.ds(_L * p, _L)]
                        ha = lax.bitcast_convert_type(hw << 16, jnp.float32)
                        hb2 = lax.bitcast_convert_type(
                            hw & jnp.int32(-65536), jnp.float32)
                        sc_v[sb, e2, pl.ds(_L * 2 * p, _L)] = ha * w
                        sc_v[sb, e2, pl.ds(_L * (2 * p + 1), _L)] = hb2 * w
                    sc_v[sb, e2, pl.ds(row_w - _L, _L)] = w
                dsc.append(pltpu.async_copy(
                    sc_v.at[sb], acc_sh.at[idxd_v.at[b]], sems.at[sb],
                    add=True))
            for d in dsc[nb - ns:]:
                d.wait()
            return carry
        lax.fori_loop(0, ngrp, group, 0)

        plsc.subcore_barrier()
        pltpu.sync_copy(acc_sh.at[pl.ds(sid * rpt, rpt)],
                        out_hbm.at[cid, pl.ds(sid * rpt, rpt)])
    return k


def _make_edge_pass(n_nodes, n_edges, row_w, n_vregs_payload, ch):
    ew = n_edges // _NW
    nch = ew // ch
    assert ew % ch == 0 and ch % 8 == 0 and ch <= 128
    npad = -(-n_nodes // (128 * _NS)) * (128 * _NS)
    rpt = npad // _NS
    assert rpt % ch == 0
    mesh = plsc.VectorSubcoreMesh(core_axis_name="c", subcore_axis_name="s",
                                  num_cores=_NC, num_subcores=_NS)

    nb = 5
    assert nch % nb == 0
    ngrp = nch // nb

    @functools.partial(
        pl.kernel,
        out_type=jax.ShapeDtypeStruct((_NC, npad, row_w), jnp.float32),
        mesh=mesh,
        compiler_params=pltpu.CompilerParams(use_tc_tiling_on_sc=False),
        scratch_types=[
            pltpu.VMEM((nb, ch), jnp.int32),
            pltpu.VMEM((nb, ch), jnp.int32),
            pltpu.VMEM((nb, ch, row_w), jnp.float32),
            pltpu.VMEM((nb, ch, _L), jnp.float32),
            pltpu.VMEM((_L,), jnp.float32),
            pltpu.VMEM_SHARED((npad, row_w), jnp.float32),
            pltpu.SemaphoreType.DMA((nb,)),
            pltpu.SemaphoreType.DMA((nb,)),
            pltpu.SemaphoreType.DMA,
        ],
    )
    def k(src_hbm, dst_hbm, pack_hbm, adp_hbm, m_hbm, out_hbm,
          idxs_v, idxd_v, rows_v, ad_v, m_v, acc_sh,
          semi, semg, sems):
        cid = lax.axis_index("c")
        sid = lax.axis_index("s")

        @plsc.parallel_loop(0, ch)
        def zrow(r):
            for kk in range(row_w // _L):
                rows_v[0, r, pl.ds(kk * _L, _L)] = jnp.zeros((_L,), jnp.float32)
        dz = []
        for b in range(rpt // ch):
            dz.append(pltpu.async_copy(
                rows_v.at[0], acc_sh.at[pl.ds(sid * rpt + b * ch, ch)], sems))
        for d in dz:
            d.wait()
        pltpu.sync_copy(m_hbm, m_v)
        plsc.subcore_barrier()

        mvec = m_v[...]
        ebase = (cid * _NS + sid) * ew

        def group(g, carry):
            base0 = ebase + g * (nb * ch)
            di = []
            for b in range(nb):
                di.append(pltpu.async_copy(
                    src_hbm.at[pl.ds(base0 + b * ch, ch)],
                    idxs_v.at[b], semi.at[b]))
                di.append(pltpu.async_copy(
                    dst_hbm.at[pl.ds(base0 + b * ch, ch)],
                    idxd_v.at[b], semi.at[b]))
            dg = []
            for b in range(nb):
                di[2 * b].wait()
                di[2 * b + 1].wait()
                dg.append(pltpu.async_copy(
                    pack_hbm.at[idxs_v.at[b]], rows_v.at[b], semg.at[b]))
                dg.append(pltpu.async_copy(
                    adp_hbm.at[idxd_v.at[b]], ad_v.at[b], semg.at[b]))
            dsc = []
            for b in range(nb):
                dg[2 * b].wait()
                dg[2 * b + 1].wait()
                @plsc.parallel_loop(0, ch, unroll=4)
                def edge(e2, b=b):
                    asv = rows_v[b, e2, pl.ds(row_w - _L, _L)]
                    t = asv + ad_v[b, e2, pl.ds(0, _L)]
                    w = jnp.exp(jnp.maximum(t, _NEG * t) - mvec)
                    for kk in range(n_vregs_payload):
                        hv = rows_v[b, e2, pl.ds(kk * _L, _L)]
                        rows_v[b, e2, pl.ds(kk * _L, _L)] = hv * w
                    rows_v[b, e2, pl.ds(row_w - _L, _L)] = w
                dsc.append(pltpu.async_copy(
                    rows_v.at[b], acc_sh.at[idxd_v.at[b]], sems, add=True))
            for d in dsc:
                d.wait()
            return carry
        lax.fori_loop(0, ngrp, group, 0)

        plsc.subcore_barrier()
        pltpu.sync_copy(acc_sh.at[pl.ds(sid * rpt, rpt)],
                        out_hbm.at[cid, pl.ds(sid * rpt, rpt)])
    return k


def kernel(x, edge_index, W1, a1s, a1d, b1, W2, a2s, a2d, b2):
    n, f_in = x.shape
    e = edge_index.shape[1]
    h, f_hid = a1s.shape
    c1 = h * f_hid
    p1 = c1 + _L
    f32 = jnp.float32

    hh, ff = np.meshgrid(np.arange(h), np.arange(f_hid), indexing="ij")
    dest = ((ff // 2) * _L + (ff % 2) * h + hh).reshape(-1)
    inv = np.empty((c1,), np.int64)
    inv[dest] = np.arange(c1)

    cc_ = np.arange(c1)
    pp_, rr_ = cc_ // (2 * _L), cc_ % (2 * _L)
    tobf = np.where(rr_ < _L, 2 * _L * pp_ + 2 * rr_,
                    2 * _L * pp_ + 2 * (rr_ - _L) + 1)
    invb = np.empty((c1,), np.int64)
    invb[tobf[dest]] = np.arange(c1)

    w1flat = jnp.transpose(W1, (1, 0, 2)).reshape(f_in, c1)
    w1perm = w1flat[:, inv]
    w1permb = w1flat[:, invb]
    avs = jnp.einsum("hif,hf->ih", W1, a1s)
    avd = jnp.einsum("hif,hf->ih", W1, a1d)
    wcat = jnp.concatenate(
        [w1perm, w1permb,
         jnp.tile(avs, (1, 2)), jnp.tile(avd, (1, 2))], axis=1)

    b1p = b1.reshape(1, c1)[:, inv]
    w2p = W2[inv, :]
    w2rep = jnp.tile(w2p, (1, _L))
    wcat2 = jnp.concatenate(
        [w2rep, w2rep * a2s[0], w2rep * a2d[0]], axis=1)

    src = edge_index[0]
    dst = edge_index[1]

    grid = (n // _BLK,)
    assert n % _BLK == 0

    hsrc, hb, asp, adp, m1 = pl.pallas_call(
        _prep_body,
        grid=grid,
        in_specs=[_row_spec(f_in), _bcast_spec(f_in, 2 * c1 + 2 * _L)],
        out_specs=[_row_spec(p1), _row_spec(c1), _row_spec(_L),
                   _row_spec(_L), _bcast_spec(2, _L)],
        out_shape=(jax.ShapeDtypeStruct((n, p1), f32),
                   jax.ShapeDtypeStruct((n, c1), jnp.bfloat16),
                   jax.ShapeDtypeStruct((n, _L), f32),
                   jax.ShapeDtypeStruct((n, _L), f32),
                   jax.ShapeDtypeStruct((2, _L), f32)),
    )(x, wcat)
    m1c = m1[0:1] + m1[1:2]
    hb32 = lax.bitcast_convert_type(hb.reshape(n, c1 // 2, 2), jnp.int32)

    part1 = _make_edge_pass_bf16(n, e, p1, 40)(
        src, dst, hb32, asp, adp, m1c.reshape(_L))

    l2s, l2d, m2 = pl.pallas_call(
        _mid_body,
        grid=grid,
        in_specs=[_row_spec(p1), _row_spec(p1), _row_spec(p1), _row_spec(_L),
                  _bcast_spec(1, _L), _bcast_spec(1, c1),
                  _bcast_spec(c1, 3 * _L)],
        out_specs=[_row_spec(2 * _L), _row_spec(_L), _bcast_spec(2, _L)],
        out_shape=(jax.ShapeDtypeStruct((n, 2 * _L), f32),
                   jax.ShapeDtypeStruct((n, _L), f32),
                   jax.ShapeDtypeStruct((2, _L), f32)),
    )(part1[0, :n], part1[1, :n], hsrc, adp, m1c, b1p, wcat2)
    m2c = m2[0:1] + m2[1:2]

    part2 = _make_edge_pass(n, e, 2 * _L, 1, 80)(
        src, dst, l2s, l2d, m2c.reshape(_L))

    out16 = pl.pallas_call(
        _fin_body,
        grid=grid,
        in_specs=[_row_spec(2 * _L), _row_spec(2 * _L), _row_spec(2 * _L),
                  _row_spec(_L), _bcast_spec(1, _L), _bcast_spec(1, 1)],
        out_specs=[_row_spec(_L)],
        out_shape=[jax.ShapeDtypeStruct((n, _L), f32)],
    )(part2[0, :n], part2[1, :n], l2s, l2d, m2c, b2.reshape(1, 1))[0]

    return out16[:, :1]
